# baseline (speedup 1.0000x reference)
import jax
import jax.numpy as jnp
from jax import lax
from jax.experimental import pallas as pl
from jax.experimental.pallas import tpu as pltpu
from jax.experimental.pallas import tpu_sc as plsc

_F32 = jnp.float32
_PROPOSALS = 1000
_PRE = 6000
_TAU = 0.7
_K = 256
_S = 6144
_NB = _S // _K
_P = 1024
_STD = (0.1, 0.1, 0.2, 0.2)

_DN = (((1,), (0,)), ((), ()))


def _colT(v):
    return jnp.swapaxes(v, 0, 1)


def _suppress_ind(cols, rows):
    cy1, cx1, cy2, cx2, car = cols
    ry1, rx1, ry2, rx2, rar = rows
    yy1 = jnp.maximum(ry1, cy1)
    xx1 = jnp.maximum(rx1, cx1)
    yy2 = jnp.minimum(ry2, cy2)
    xx2 = jnp.minimum(rx2, cx2)
    inter = jnp.maximum(yy2 - yy1, 0.0) * jnp.maximum(xx2 - xx1, 0.0)
    iou = inter / (rar + car - inter + 1e-8)
    return iou > _TAU


def _nms_body(at_ref, dt_ref, out_ref, crd, keeps):
    a = at_ref[0]
    d = dt_ref[0]
    a0, a1, a2, a3 = a[0:1], a[1:2], a[2:3], a[3:4]
    d0, d1, d2, d3 = (d[0:1] * _STD[0], d[1:2] * _STD[1],
                      d[2:3] * _STD[2], d[3:4] * _STD[3])
    h = a2 - a0
    w = a3 - a1
    cy = (a0 + 0.5 * h) + d0 * h
    cx = (a1 + 0.5 * w) + d1 * w
    he = h * jnp.exp(d2)
    we = w * jnp.exp(d3)
    y1 = cy - 0.5 * he
    x1 = cx - 0.5 * we
    y2 = y1 + he
    x2 = x1 + we
    y1 = jnp.clip(y1, 0.0, 1.0)
    x1 = jnp.clip(x1, 0.0, 1.0)
    y2 = jnp.clip(y2, 0.0, 1.0)
    x2 = jnp.clip(x2, 0.0, 1.0)
    ar = (y2 - y1) * (x2 - x1)
    crd[0:1, :] = y1
    crd[1:2, :] = x1
    crd[2:3, :] = y2
    crd[3:4, :] = x2
    crd[4:5, :] = ar

    out_ref[0] = jnp.zeros((4, _P), _F32)

    io_s = lax.broadcasted_iota(jnp.int32, (_K, _K), 0)
    io_l = lax.broadcasted_iota(jnp.int32, (_K, _K), 1)
    lt = (io_l < io_s).astype(_F32)
    p_row = lax.broadcasted_iota(jnp.int32, (1, _P), 1)
    blk_iota = lax.broadcasted_iota(jnp.int32, (_K, 1), 0)

    def get_rows(base):
        sl = pl.ds(base, _K)
        return (crd[0:1, sl], crd[1:2, sl], crd[2:3, sl], crd[3:4, sl],
                crd[4:5, sl])

    def process(b, kc):
        base = pl.multiple_of(b * _K, _K)
        rows_b = get_rows(base)
        cols_b = tuple(_colT(r) for r in rows_b)

        def cross(c, acc):
            cb = pl.multiple_of(c * _K, _K)
            ind = _suppress_ind(cols_b, get_rows(cb)).astype(_F32)
            kcol = keeps[pl.ds(cb, _K), :]
            return acc + lax.dot_general(ind, kcol, _DN,
                                         preferred_element_type=_F32)

        sup = lax.fori_loop(0, b, cross, jnp.zeros((_K, 1), _F32))
        cand = ((blk_iota + base) < _PRE).astype(_F32)
        base_keep = cand * (sup < 0.5).astype(_F32)

        mb = _suppress_ind(cols_b, rows_b).astype(_F32) * lt

        def fix_cond(st):
            return st[1]

        def fix_body(st):
            k = st[0]
            s = lax.dot_general(mb, k, _DN, preferred_element_type=_F32)
            nk = base_keep * (s < 0.5).astype(_F32)
            return nk, jnp.any(nk != k)

        keep_b, _ = lax.while_loop(fix_cond, fix_body, (base_keep, True))
        keeps[pl.ds(base, _K), :] = keep_b

        rank = lax.dot_general(lt, keep_b, _DN, preferred_element_type=_F32)
        pos = (kc + rank).astype(jnp.int32)
        sel = ((pos == p_row) & (keep_b > 0.5)).astype(_F32)
        boxr = jnp.concatenate(rows_b[:4], axis=0)
        out_ref[0] += lax.dot_general(boxr, sel, _DN,
                                      preferred_element_type=_F32)
        return kc + jnp.sum(keep_b)

    def blk(b, kc):
        return lax.cond(kc < float(_PROPOSALS), process,
                        lambda b_, k_: k_, b, kc)

    lax.fori_loop(0, _NB, blk, 0.0)


_NP = 20480
_NV = _NP // 16
_RAD = 1024
_KEYTOP = 0x3FFFFFFF


def _sc_sort_gather_body(s_hbm, a_hbm, d_hbm, aout, dout, k0, i0, k1,
                         i1, hist, offs, idxh, rows, sem):
    cid = lax.axis_index("c")
    sid = lax.axis_index("s")
    io16 = lax.iota(jnp.int32, 16)

    @pl.when(sid == 0)
    def _():
        b = cid
        pltpu.sync_copy(s_hbm.at[b], k0)

        def build(i, _):
            i0[pl.ds(i * 16, 16)] = io16 + i * 16
            return 0

        lax.fori_loop(0, _NV, build, 0, unroll=8)

        for p, (kin, iin, kout, iout) in enumerate(
                ((k0, i0, k1, i1), (k1, i1, k0, i0), (k0, i0, k1, i1))):
            shift = 10 * p

            def zero(i, _):
                hist[pl.ds(i * 16, 16)] = jnp.zeros((16,), jnp.int32)
                return 0

            lax.fori_loop(0, _RAD // 16, zero, 0)

            def histo(i, _):
                k = kin[pl.ds(i * 16, 16)]
                dg = lax.shift_right_logical(k, shift) & (_RAD - 1)
                occ, last = plsc.scan_count(dg)
                plsc.addupdate_scatter(hist, [dg], occ, mask=last)
                return 0

            lax.fori_loop(0, _NV, histo, 0, unroll=4)

            def scan(i, carry):
                h = hist[pl.ds(i * 16, 16)]
                inc = plsc.cumsum(h)
                offs[pl.ds(i * 16, 16)] = (inc - h) + carry
                return carry + jnp.sum(h)

            lax.fori_loop(0, _RAD // 16, scan, 0)

            def permute(i, _):
                o = i * 16
                k = kin[pl.ds(o, 16)]
                v = iin[pl.ds(o, 16)]
                dg = lax.shift_right_logical(k, shift) & (_RAD - 1)
                occ, last = plsc.scan_count(dg)
                base = plsc.load_gather(offs, [dg])
                slot = base + occ - 1
                plsc.store_scatter(kout, [slot], k)
                plsc.store_scatter(iout, [slot], v)
                plsc.addupdate_scatter(offs, [dg], occ, mask=last)
                return 0

            lax.fori_loop(0, _NV, permute, 0, unroll=4)

        def cp(i, _):
            i16 = i * 16
            idxh[pl.ds(i16, 16)] = i1[pl.ds(i16, 16)]
            return 0

        lax.fori_loop(0, _S // 16, cp, 0, unroll=8)
        for src_hbm, dst_hbm in ((a_hbm, aout), (d_hbm, dout)):
            copies = [
                pltpu.async_copy(src_hbm.at[b].at[c].at[idxh], rows[c], sem)
                for c in range(4)
            ]
            for c in range(4):
                copies[c].wait()
                pltpu.sync_copy(rows[c], dst_hbm.at[b].at[c])


def _sc_sort_gather(fg_pad, anchors, deltas):
    B = fg_pad.shape[0]
    f = pl.kernel(
        _sc_sort_gather_body,
        out_type=(
            jax.ShapeDtypeStruct((B, 4, _S), _F32),
            jax.ShapeDtypeStruct((B, 4, _S), _F32),
        ),
        mesh=plsc.VectorSubcoreMesh(core_axis_name="c", subcore_axis_name="s"),
        scratch_types=[
            pltpu.VMEM((_NP,), jnp.int32),
            pltpu.VMEM((_NP,), jnp.int32),
            pltpu.VMEM((_NP,), jnp.int32),
            pltpu.VMEM((_NP,), jnp.int32),
            pltpu.VMEM((_RAD,), jnp.int32),
            pltpu.VMEM((_RAD,), jnp.int32),
            pltpu.VMEM((_S,), jnp.int32),
            [pltpu.VMEM((_S,), _F32)] * 4,
            pltpu.SemaphoreType.DMA,
        ],
        compiler_params=pltpu.CompilerParams(needs_layout_passes=False,
                                             use_tc_tiling_on_sc=False),
    )
    return f(fg_pad, anchors, deltas)


def kernel(scores, deltas, anchors):
    B, N, _ = scores.shape
    fg = jnp.pad(scores[:, :, 1], ((0, 0), (0, _NP - N)))
    keys = _KEYTOP - lax.bitcast_convert_type(fg, jnp.int32)
    at, dt = _sc_sort_gather(keys, jnp.transpose(anchors, (0, 2, 1)),
                             jnp.transpose(deltas, (0, 2, 1)))

    out = pl.pallas_call(
        _nms_body,
        grid=(B,),
        in_specs=[
            pl.BlockSpec((1, 4, _S), lambda b: (b, 0, 0)),
            pl.BlockSpec((1, 4, _S), lambda b: (b, 0, 0)),
        ],
        out_specs=pl.BlockSpec((1, 4, _P), lambda b: (b, 0, 0)),
        out_shape=jax.ShapeDtypeStruct((B, 4, _P), _F32),
        scratch_shapes=[
            pltpu.VMEM((8, _S), _F32),
            pltpu.VMEM((_S, 1), _F32),
        ],
    )(at, dt)
    return jnp.transpose(out, (0, 2, 1))[:, :_PROPOSALS, :]


# speedup vs baseline: 1.2341x; 1.2341x over previous
import jax
import jax.numpy as jnp
from jax import lax
from jax.experimental import pallas as pl
from jax.experimental.pallas import tpu as pltpu
from jax.experimental.pallas import tpu_sc as plsc

_F32 = jnp.float32
_PROPOSALS = 1000
_PRE = 6000
_TAU = 0.7
_K = 256
_S = 6144
_NB = _S // _K
_P = 1024
_STD = (0.1, 0.1, 0.2, 0.2)

_DN = (((1,), (0,)), ((), ()))


def _colT(v):
    return jnp.swapaxes(v, 0, 1)


def _suppress_ind(cols, rows):
    cy1, cx1, cy2, cx2, car = cols
    ry1, rx1, ry2, rx2, rar = rows
    yy1 = jnp.maximum(ry1, cy1)
    xx1 = jnp.maximum(rx1, cx1)
    yy2 = jnp.minimum(ry2, cy2)
    xx2 = jnp.minimum(rx2, cx2)
    inter = jnp.maximum(yy2 - yy1, 0.0) * jnp.maximum(xx2 - xx1, 0.0)
    iou = inter / (rar + car - inter + 1e-8)
    return iou > _TAU


def _nms_body(at_ref, dt_ref, out_ref, crd, keeps):
    a = at_ref[0]
    d = dt_ref[0]
    a0, a1, a2, a3 = a[0:1], a[1:2], a[2:3], a[3:4]
    d0, d1, d2, d3 = (d[0:1] * _STD[0], d[1:2] * _STD[1],
                      d[2:3] * _STD[2], d[3:4] * _STD[3])
    h = a2 - a0
    w = a3 - a1
    cy = (a0 + 0.5 * h) + d0 * h
    cx = (a1 + 0.5 * w) + d1 * w
    he = h * jnp.exp(d2)
    we = w * jnp.exp(d3)
    y1 = cy - 0.5 * he
    x1 = cx - 0.5 * we
    y2 = y1 + he
    x2 = x1 + we
    y1 = jnp.clip(y1, 0.0, 1.0)
    x1 = jnp.clip(x1, 0.0, 1.0)
    y2 = jnp.clip(y2, 0.0, 1.0)
    x2 = jnp.clip(x2, 0.0, 1.0)
    ar = (y2 - y1) * (x2 - x1)
    crd[0:1, :] = y1
    crd[1:2, :] = x1
    crd[2:3, :] = y2
    crd[3:4, :] = x2
    crd[4:5, :] = ar

    out_ref[0] = jnp.zeros((4, _P), _F32)

    io_s = lax.broadcasted_iota(jnp.int32, (_K, _K), 0)
    io_l = lax.broadcasted_iota(jnp.int32, (_K, _K), 1)
    lt = (io_l < io_s).astype(_F32)
    p_row = lax.broadcasted_iota(jnp.int32, (1, _P), 1)
    blk_iota = lax.broadcasted_iota(jnp.int32, (_K, 1), 0)

    def get_rows(base):
        sl = pl.ds(base, _K)
        return (crd[0:1, sl], crd[1:2, sl], crd[2:3, sl], crd[3:4, sl],
                crd[4:5, sl])

    def process(b, kc):
        base = pl.multiple_of(b * _K, _K)
        rows_b = get_rows(base)
        cols_b = tuple(_colT(r) for r in rows_b)

        def cross(c, acc):
            cb = pl.multiple_of(c * _K, _K)
            ind = _suppress_ind(cols_b, get_rows(cb)).astype(_F32)
            kcol = keeps[pl.ds(cb, _K), :]
            return acc + lax.dot_general(ind, kcol, _DN,
                                         preferred_element_type=_F32)

        sup = lax.fori_loop(0, b, cross, jnp.zeros((_K, 1), _F32))
        cand = ((blk_iota + base) < _PRE).astype(_F32)
        base_keep = cand * (sup < 0.5).astype(_F32)

        mb = _suppress_ind(cols_b, rows_b).astype(_F32) * lt

        def fix_cond(st):
            return st[1]

        def fix_body(st):
            k = st[0]
            s = lax.dot_general(mb, k, _DN, preferred_element_type=_F32)
            nk = base_keep * (s < 0.5).astype(_F32)
            return nk, jnp.any(nk != k)

        keep_b, _ = lax.while_loop(fix_cond, fix_body, (base_keep, True))
        keeps[pl.ds(base, _K), :] = keep_b

        rank = lax.dot_general(lt, keep_b, _DN, preferred_element_type=_F32)
        pos = (kc + rank).astype(jnp.int32)
        sel = ((pos == p_row) & (keep_b > 0.5)).astype(_F32)
        boxr = jnp.concatenate(rows_b[:4], axis=0)
        out_ref[0] += lax.dot_general(boxr, sel, _DN,
                                      preferred_element_type=_F32)
        return kc + jnp.sum(keep_b)

    def blk(b, kc):
        return lax.cond(kc < float(_PROPOSALS), process,
                        lambda b_, k_: k_, b, kc)

    lax.fori_loop(0, _NB, blk, 0.0)


_NP = 20480
_NV = _NP // 16
_RAD = 1024
_KEYTOP = 0x3FFFFFFF


def _sc_sort_gather_body(s_hbm, a_hbm, d_hbm, aout, dout, k0, i0, k1,
                         i1, hist, offs, idxh, rows, sem):
    cid = lax.axis_index("c")
    sid = lax.axis_index("s")
    io16 = lax.iota(jnp.int32, 16)

    @pl.when(sid == 0)
    def _():
        b = cid
        pltpu.sync_copy(s_hbm.at[b], k0)

        def build(i, _):
            i0[pl.ds(i * 16, 16)] = io16 + i * 16
            return 0

        lax.fori_loop(0, _NV, build, 0, unroll=8)

        def zero(i, _):
            hist[pl.ds(i * 16, 16)] = jnp.zeros((16,), jnp.int32)
            return 0

        lax.fori_loop(0, _RAD // 16, zero, 0)

        def histo_top(i, _):
            k = k0[pl.ds(i * 16, 16)]
            dg = lax.shift_right_logical(k, 20) & (_RAD - 1)
            occ, last = plsc.scan_count(dg)
            plsc.addupdate_scatter(hist, [dg], occ, mask=last)
            return 0

        lax.fori_loop(0, _NV, histo_top, 0, unroll=4)

        def scan(i, carry):
            h = hist[pl.ds(i * 16, 16)]
            inc = plsc.cumsum(h)
            offs[pl.ds(i * 16, 16)] = (inc - h) + carry
            return carry + jnp.sum(h)

        lax.fori_loop(0, _RAD // 16, scan, 0)

        def cutoff(i, g):
            o = i * 16
            ex = offs[pl.ds(o, 16)]
            h = hist[pl.ds(o, 16)]
            m = (ex < _S) & ((ex + h) >= _S)
            cand = jnp.where(m, io16 + o, _RAD)
            return jnp.minimum(g, jnp.min(cand))

        gstar = lax.fori_loop(0, _RAD // 16, cutoff, _RAD)

        def prefill(i, _):
            k1[pl.ds(i * 16, 16)] = jnp.full((16,), 0x7FFFFFFF, jnp.int32)
            return 0

        lax.fori_loop(0, _NV, prefill, 0, unroll=8)

        def compact(i, w):
            o = i * 16
            k = k0[pl.ds(o, 16)]
            v = i0[pl.ds(o, 16)]
            dg = lax.shift_right_logical(k, 20) & (_RAD - 1)
            m = dg <= gstar
            mi = jnp.where(m, 1, 0)
            slot = w + plsc.cumsum(mi) - 1
            plsc.store_scatter(k1, [slot], k, mask=m)
            plsc.store_scatter(i1, [slot], v, mask=m)
            return w + jnp.sum(mi)

        mcnt = lax.fori_loop(0, _NV, compact, 0, unroll=4)
        nv = (mcnt + 15) // 16

        for p, (kin, iin, kout, iout) in enumerate(
                ((k1, i1, k0, i0), (k0, i0, k1, i1), (k1, i1, k0, i0))):
            shift = 10 * p

            lax.fori_loop(0, _RAD // 16, zero, 0)

            def histo(i, _):
                k = kin[pl.ds(i * 16, 16)]
                dg = lax.shift_right_logical(k, shift) & (_RAD - 1)
                occ, last = plsc.scan_count(dg)
                plsc.addupdate_scatter(hist, [dg], occ, mask=last)
                return 0

            lax.fori_loop(0, nv, histo, 0)

            lax.fori_loop(0, _RAD // 16, scan, 0)

            def permute(i, _):
                o = i * 16
                k = kin[pl.ds(o, 16)]
                v = iin[pl.ds(o, 16)]
                dg = lax.shift_right_logical(k, shift) & (_RAD - 1)
                occ, last = plsc.scan_count(dg)
                base = plsc.load_gather(offs, [dg])
                slot = base + occ - 1
                plsc.store_scatter(kout, [slot], k)
                plsc.store_scatter(iout, [slot], v)
                plsc.addupdate_scatter(offs, [dg], occ, mask=last)
                return 0

            lax.fori_loop(0, nv, permute, 0)

        def cp(i, _):
            i16 = i * 16
            idxh[pl.ds(i16, 16)] = i0[pl.ds(i16, 16)]
            return 0

        lax.fori_loop(0, _S // 16, cp, 0, unroll=8)
        for src_hbm, dst_hbm in ((a_hbm, aout), (d_hbm, dout)):
            copies = [
                pltpu.async_copy(src_hbm.at[b].at[c].at[idxh], rows[c], sem)
                for c in range(4)
            ]
            for c in range(4):
                copies[c].wait()
                pltpu.sync_copy(rows[c], dst_hbm.at[b].at[c])


def _sc_sort_gather(fg_pad, anchors, deltas):
    B = fg_pad.shape[0]
    f = pl.kernel(
        _sc_sort_gather_body,
        out_type=(
            jax.ShapeDtypeStruct((B, 4, _S), _F32),
            jax.ShapeDtypeStruct((B, 4, _S), _F32),
        ),
        mesh=plsc.VectorSubcoreMesh(core_axis_name="c", subcore_axis_name="s"),
        scratch_types=[
            pltpu.VMEM((_NP,), jnp.int32),
            pltpu.VMEM((_NP,), jnp.int32),
            pltpu.VMEM((_NP,), jnp.int32),
            pltpu.VMEM((_NP,), jnp.int32),
            pltpu.VMEM((_RAD,), jnp.int32),
            pltpu.VMEM((_RAD,), jnp.int32),
            pltpu.VMEM((_S,), jnp.int32),
            [pltpu.VMEM((_S,), _F32)] * 4,
            pltpu.SemaphoreType.DMA,
        ],
        compiler_params=pltpu.CompilerParams(needs_layout_passes=False,
                                             use_tc_tiling_on_sc=False),
    )
    return f(fg_pad, anchors, deltas)


def kernel(scores, deltas, anchors):
    B, N, _ = scores.shape
    fg = jnp.pad(scores[:, :, 1], ((0, 0), (0, _NP - N)))
    keys = _KEYTOP - lax.bitcast_convert_type(fg, jnp.int32)
    at, dt = _sc_sort_gather(keys, jnp.transpose(anchors, (0, 2, 1)),
                             jnp.transpose(deltas, (0, 2, 1)))

    out = pl.pallas_call(
        _nms_body,
        grid=(B,),
        in_specs=[
            pl.BlockSpec((1, 4, _S), lambda b: (b, 0, 0)),
            pl.BlockSpec((1, 4, _S), lambda b: (b, 0, 0)),
        ],
        out_specs=pl.BlockSpec((1, 4, _P), lambda b: (b, 0, 0)),
        out_shape=jax.ShapeDtypeStruct((B, 4, _P), _F32),
        scratch_shapes=[
            pltpu.VMEM((8, _S), _F32),
            pltpu.VMEM((_S, 1), _F32),
        ],
    )(at, dt)
    return jnp.transpose(out, (0, 2, 1))[:, :_PROPOSALS, :]


# speedup vs baseline: 1.2404x; 1.0052x over previous
import jax
import jax.numpy as jnp
from jax import lax
from jax.experimental import pallas as pl
from jax.experimental.pallas import tpu as pltpu
from jax.experimental.pallas import tpu_sc as plsc

_F32 = jnp.float32
_PROPOSALS = 1000
_PRE = 6000
_TAU = 0.7
_K = 256
_S = 6144
_NB = _S // _K
_P = 1024
_STD = (0.1, 0.1, 0.2, 0.2)

_DN = (((1,), (0,)), ((), ()))


def _colT(v):
    return jnp.swapaxes(v, 0, 1)


def _suppress_ind(cols, rows):
    cy1, cx1, cy2, cx2, car = cols
    ry1, rx1, ry2, rx2, rar = rows
    yy1 = jnp.maximum(ry1, cy1)
    xx1 = jnp.maximum(rx1, cx1)
    yy2 = jnp.minimum(ry2, cy2)
    xx2 = jnp.minimum(rx2, cx2)
    inter = jnp.maximum(yy2 - yy1, 0.0) * jnp.maximum(xx2 - xx1, 0.0)
    iou = inter / (rar + car - inter + 1e-8)
    return iou > _TAU


def _nms_body(at_ref, dt_ref, out_ref, crd, keeps):
    a = at_ref[0]
    d = dt_ref[0]
    a0, a1, a2, a3 = a[0:1], a[1:2], a[2:3], a[3:4]
    d0, d1, d2, d3 = (d[0:1] * _STD[0], d[1:2] * _STD[1],
                      d[2:3] * _STD[2], d[3:4] * _STD[3])
    h = a2 - a0
    w = a3 - a1
    cy = (a0 + 0.5 * h) + d0 * h
    cx = (a1 + 0.5 * w) + d1 * w
    he = h * jnp.exp(d2)
    we = w * jnp.exp(d3)
    y1 = cy - 0.5 * he
    x1 = cx - 0.5 * we
    y2 = y1 + he
    x2 = x1 + we
    y1 = jnp.clip(y1, 0.0, 1.0)
    x1 = jnp.clip(x1, 0.0, 1.0)
    y2 = jnp.clip(y2, 0.0, 1.0)
    x2 = jnp.clip(x2, 0.0, 1.0)
    ar = (y2 - y1) * (x2 - x1)
    crd[0:1, :] = y1
    crd[1:2, :] = x1
    crd[2:3, :] = y2
    crd[3:4, :] = x2
    crd[4:5, :] = ar

    out_ref[0] = jnp.zeros((4, _P), _F32)

    io_s = lax.broadcasted_iota(jnp.int32, (_K, _K), 0)
    io_l = lax.broadcasted_iota(jnp.int32, (_K, _K), 1)
    lt = (io_l < io_s).astype(_F32)
    p_row = lax.broadcasted_iota(jnp.int32, (1, _P), 1)
    blk_iota = lax.broadcasted_iota(jnp.int32, (_K, 1), 0)

    def get_rows(base):
        sl = pl.ds(base, _K)
        return (crd[0:1, sl], crd[1:2, sl], crd[2:3, sl], crd[3:4, sl],
                crd[4:5, sl])

    def process(b, kc):
        base = pl.multiple_of(b * _K, _K)
        rows_b = get_rows(base)
        cols_b = tuple(_colT(r) for r in rows_b)

        def cross(c, acc):
            cb = pl.multiple_of(c * _K, _K)
            ind = _suppress_ind(cols_b, get_rows(cb)).astype(_F32)
            kcol = keeps[pl.ds(cb, _K), :]
            return acc + lax.dot_general(ind, kcol, _DN,
                                         preferred_element_type=_F32)

        sup = lax.fori_loop(0, b, cross, jnp.zeros((_K, 1), _F32))
        cand = ((blk_iota + base) < _PRE).astype(_F32)
        base_keep = cand * (sup < 0.5).astype(_F32)

        mb = _suppress_ind(cols_b, rows_b).astype(_F32) * lt

        def fix_cond(st):
            return st[1]

        def fix_body(st):
            k = st[0]
            s = lax.dot_general(mb, k, _DN, preferred_element_type=_F32)
            nk = base_keep * (s < 0.5).astype(_F32)
            return nk, jnp.any(nk != k)

        keep_b, _ = lax.while_loop(fix_cond, fix_body, (base_keep, True))
        keeps[pl.ds(base, _K), :] = keep_b

        rank = lax.dot_general(lt, keep_b, _DN, preferred_element_type=_F32)
        pos = (kc + rank).astype(jnp.int32)
        sel = ((pos == p_row) & (keep_b > 0.5)).astype(_F32)
        boxr = jnp.concatenate(rows_b[:4], axis=0)
        out_ref[0] += lax.dot_general(boxr, sel, _DN,
                                      preferred_element_type=_F32)
        return kc + jnp.sum(keep_b)

    def blk(b, kc):
        return lax.cond(kc < float(_PROPOSALS), process,
                        lambda b_, k_: k_, b, kc)

    lax.fori_loop(0, _NB, blk, 0.0)


_NP = 20480
_NV = _NP // 16
_RAD = 1024
_KEYTOP = 0x3FFFFFFF


def _sc_sort_gather_body(s_hbm, a_hbm, d_hbm, aout, dout, k0, i0, k1,
                         i1, hist, offs, idxh, rows, sem):
    cid = lax.axis_index("c")
    sid = lax.axis_index("s")
    io16 = lax.iota(jnp.int32, 16)

    @pl.when(sid == 0)
    def _():
        b = cid
        pltpu.sync_copy(s_hbm.at[b], k0)

        def zero(i, _):
            hist[pl.ds(i * 16, 16)] = jnp.zeros((16,), jnp.int32)
            return 0

        lax.fori_loop(0, _RAD // 16, zero, 0)

        def histo_top(i, _):
            k = k0[pl.ds(i * 16, 16)]
            dg = lax.shift_right_logical(k, 20) & (_RAD - 1)
            occ, last = plsc.scan_count(dg)
            plsc.addupdate_scatter(hist, [dg], occ, mask=last)
            return 0

        lax.fori_loop(0, _NV, histo_top, 0, unroll=4)

        def scan(i, carry):
            h = hist[pl.ds(i * 16, 16)]
            inc = plsc.cumsum(h)
            offs[pl.ds(i * 16, 16)] = (inc - h) + carry
            return carry + jnp.sum(h)

        lax.fori_loop(0, _RAD // 16, scan, 0)

        def cutoff(i, g):
            o = i * 16
            ex = offs[pl.ds(o, 16)]
            h = hist[pl.ds(o, 16)]
            m = (ex < _S) & ((ex + h) >= _S)
            cand = jnp.where(m, io16 + o, _RAD)
            return jnp.minimum(g, jnp.min(cand))

        gstar = lax.fori_loop(0, _RAD // 16, cutoff, _RAD)

        def compact(i, w):
            o = i * 16
            k = k0[pl.ds(o, 16)]
            dg = lax.shift_right_logical(k, 20) & (_RAD - 1)
            m = dg <= gstar
            mi = jnp.where(m, 1, 0)
            slot = w + plsc.cumsum(mi) - 1
            plsc.store_scatter(k1, [slot], k, mask=m)
            plsc.store_scatter(i1, [slot], io16 + o, mask=m)
            return w + jnp.sum(mi)

        mcnt = lax.fori_loop(0, _NV, compact, 0, unroll=4)
        nv = (mcnt + 15) // 16
        tail = io16 + (nv - 1) * 16
        plsc.store_scatter(k1, [tail], jnp.full((16,), 0x7FFFFFFF,
                                                jnp.int32),
                           mask=tail >= mcnt)

        for p, (kin, iin, kout, iout) in enumerate(
                ((k1, i1, k0, i0), (k0, i0, k1, i1), (k1, i1, k0, i0))):
            shift = 10 * p

            lax.fori_loop(0, _RAD // 16, zero, 0)

            def histo(i, _):
                k = kin[pl.ds(i * 16, 16)]
                dg = lax.shift_right_logical(k, shift) & (_RAD - 1)
                occ, last = plsc.scan_count(dg)
                plsc.addupdate_scatter(hist, [dg], occ, mask=last)
                return 0

            lax.fori_loop(0, _S // 16, histo, 0, unroll=4)
            lax.fori_loop(_S // 16, nv, histo, 0)

            lax.fori_loop(0, _RAD // 16, scan, 0)

            def permute(i, _):
                o = i * 16
                k = kin[pl.ds(o, 16)]
                v = iin[pl.ds(o, 16)]
                dg = lax.shift_right_logical(k, shift) & (_RAD - 1)
                occ, last = plsc.scan_count(dg)
                base = plsc.load_gather(offs, [dg])
                slot = base + occ - 1
                plsc.store_scatter(kout, [slot], k)
                plsc.store_scatter(iout, [slot], v)
                plsc.addupdate_scatter(offs, [dg], occ, mask=last)
                return 0

            lax.fori_loop(0, _S // 16, permute, 0, unroll=4)
            lax.fori_loop(_S // 16, nv, permute, 0)

        def cp(i, _):
            i16 = i * 16
            idxh[pl.ds(i16, 16)] = i0[pl.ds(i16, 16)]
            return 0

        lax.fori_loop(0, _S // 16, cp, 0, unroll=8)
        for src_hbm, dst_hbm in ((a_hbm, aout), (d_hbm, dout)):
            copies = [
                pltpu.async_copy(src_hbm.at[b].at[c].at[idxh], rows[c], sem)
                for c in range(4)
            ]
            for c in range(4):
                copies[c].wait()
                pltpu.sync_copy(rows[c], dst_hbm.at[b].at[c])


def _sc_sort_gather(fg_pad, anchors, deltas):
    B = fg_pad.shape[0]
    f = pl.kernel(
        _sc_sort_gather_body,
        out_type=(
            jax.ShapeDtypeStruct((B, 4, _S), _F32),
            jax.ShapeDtypeStruct((B, 4, _S), _F32),
        ),
        mesh=plsc.VectorSubcoreMesh(core_axis_name="c", subcore_axis_name="s"),
        scratch_types=[
            pltpu.VMEM((_NP,), jnp.int32),
            pltpu.VMEM((_NP,), jnp.int32),
            pltpu.VMEM((_NP,), jnp.int32),
            pltpu.VMEM((_NP,), jnp.int32),
            pltpu.VMEM((_RAD,), jnp.int32),
            pltpu.VMEM((_RAD,), jnp.int32),
            pltpu.VMEM((_S,), jnp.int32),
            [pltpu.VMEM((_S,), _F32)] * 4,
            pltpu.SemaphoreType.DMA,
        ],
        compiler_params=pltpu.CompilerParams(needs_layout_passes=False,
                                             use_tc_tiling_on_sc=False),
    )
    return f(fg_pad, anchors, deltas)


def kernel(scores, deltas, anchors):
    B, N, _ = scores.shape
    fg = jnp.pad(scores[:, :, 1], ((0, 0), (0, _NP - N)))
    keys = _KEYTOP - lax.bitcast_convert_type(fg, jnp.int32)
    at, dt = _sc_sort_gather(keys, jnp.transpose(anchors, (0, 2, 1)),
                             jnp.transpose(deltas, (0, 2, 1)))

    out = pl.pallas_call(
        _nms_body,
        grid=(B,),
        in_specs=[
            pl.BlockSpec((1, 4, _S), lambda b: (b, 0, 0)),
            pl.BlockSpec((1, 4, _S), lambda b: (b, 0, 0)),
        ],
        out_specs=pl.BlockSpec((1, 4, _P), lambda b: (b, 0, 0)),
        out_shape=jax.ShapeDtypeStruct((B, 4, _P), _F32),
        scratch_shapes=[
            pltpu.VMEM((8, _S), _F32),
            pltpu.VMEM((_S, 1), _F32),
        ],
    )(at, dt)
    return jnp.transpose(out, (0, 2, 1))[:, :_PROPOSALS, :]
